# baseline (device time: 334689 ns/iter reference)
import jax
import jax.numpy as jnp
from jax import lax
from jax.experimental import pallas as pl
from jax.experimental.pallas import tpu as pltpu

jax.config.update("jax_compilation_cache_dir", "/tmp/jax_cache")
jax.config.update("jax_persistent_cache_min_compile_time_secs", 0.0)

N_DEV = 4
M_BLK = 2048
MH = 1024
MQ = 512
NT = 512
N_TILES = 8

_SRC_OFF = (0, 3, 2, 1)
_SEM_FOR = (None, 0, 1, 2)


def kernel(x, w_mat):
    k_tot, k_loc = x.shape
    n_out = w_mat.shape[1]
    assert k_loc == M_BLK and n_out == N_TILES * NT

    def body(x_ref, w_ref, out_ref, xg_ref, stage_ref,
             cf32, cbf, xh_f32, x_half, w_buf, acc_ref,
             send_sems, recv_sems, cin_sem, cout_sem, x_sem, w_sems,
             out_sems):
        my = lax.axis_index("i")

        def src_of(j):
            return lax.rem(my + _SRC_OFF[j], N_DEV)

        barrier = pltpu.get_barrier_semaphore()
        for d in (1, 2, 3):
            t = lax.rem(my + d, N_DEV)
            pl.semaphore_signal(
                barrier, inc=1, device_id=(t,),
                device_id_type=pl.DeviceIdType.MESH,
            )
        pl.semaphore_wait(barrier, N_DEV - 1)

        sends = []
        for d in (1, 2, 3):
            t = lax.rem(my + d, N_DEV)

            def stage_q(q, t=t, d=d):
                cin = pltpu.make_async_copy(
                    x_ref.at[pl.ds(t * M_BLK + q * MQ, MQ), :],
                    cf32, cin_sem,
                )
                cin.start()
                cin.wait()
                cbf[:, :] = cf32[:, :].astype(jnp.bfloat16)
                cout = pltpu.make_async_copy(
                    cbf, stage_ref.at[d - 1, pl.ds(q * MQ, MQ), :],
                    cout_sem,
                )
                cout.start()
                cout.wait()

            def fire(row0, nrows, col, t=t, d=d):
                rdma = pltpu.make_async_remote_copy(
                    src_ref=stage_ref.at[d - 1, pl.ds(row0, nrows), :],
                    dst_ref=xg_ref.at[my, pl.ds(row0, nrows), :],
                    send_sem=send_sems.at[d - 1, col],
                    recv_sem=recv_sems.at[d - 1, col],
                    device_id=(t,),
                    device_id_type=pl.DeviceIdType.MESH,
                )
                rdma.start()
                sends.append(rdma)

            if d == 3:
                for q in (0, 1, 2, 3):
                    stage_q(q)
                    fire(q * MQ, MQ, q)
            else:
                for h in (0, 1):
                    stage_q(2 * h)
                    stage_q(2 * h + 1)
                    fire(h * MH, MH, 2 * h)

        def wait_recv(j, col, row0, nrows):
            s = src_of(j)
            sem = _SEM_FOR[j]
            pltpu.make_async_remote_copy(
                src_ref=stage_ref.at[0, pl.ds(row0, nrows), :],
                dst_ref=xg_ref.at[s, pl.ds(row0, nrows), :],
                send_sem=send_sems.at[sem, col],
                recv_sem=recv_sems.at[sem, col],
                device_id=(s,),
                device_id_type=pl.DeviceIdType.MESH,
            ).wait_recv()

        def wdma_start(s, n, slot):
            pltpu.make_async_copy(
                w_ref.at[pl.ds(s * M_BLK, M_BLK), pl.ds(n * NT, NT)],
                w_buf.at[slot], w_sems.at[slot],
            ).start()

        def wdma_wait(slot):
            pltpu.make_async_copy(
                w_ref.at[pl.ds(0, M_BLK), pl.ds(0, NT)],
                w_buf.at[slot], w_sems.at[slot],
            ).wait()

        s0 = src_of(0)
        for h in (0, 1):
            xh = pltpu.make_async_copy(
                x_ref.at[pl.ds(my * M_BLK + h * MH, MH), :], xh_f32, x_sem
            )
            xh.start()
            wdma_start(s0, 0, 0)
            wdma_start(s0, 1, 1)
            xh.wait()
            row = slice(h * MH, (h + 1) * MH)

            def j0_body(n, _, row=row):
                slot = lax.rem(n, 2)
                wdma_wait(slot)
                prod = jnp.dot(
                    xh_f32[:, :], w_buf[slot],
                    preferred_element_type=jnp.float32,
                )
                acc_ref[row, pl.ds(n * NT, NT)] = prod

                @pl.when(n + 2 < N_TILES)
                def _():
                    wdma_start(s0, n + 2, slot)

                return 0

            lax.fori_loop(0, N_TILES, j0_body, 0)

        for j in (1, 2):
            s = src_of(j)
            for h in (0, 1):
                wdma_start(s, 0, 0)
                wdma_start(s, 1, 1)
                wait_recv(j, 2 * h, h * MH, MH)
                xh = pltpu.make_async_copy(
                    xg_ref.at[s, pl.ds(h * MH, MH), :], x_half, x_sem
                )
                xh.start()
                xh.wait()
                row = slice(h * MH, (h + 1) * MH)

                def jh_body(n, _, j=j, s=s, row=row):
                    slot = lax.rem(n, 2)
                    wdma_wait(slot)
                    prod = jnp.dot(
                        x_half[:, :], w_buf[slot].astype(jnp.bfloat16),
                        preferred_element_type=jnp.float32,
                    )
                    acc_ref[row, pl.ds(n * NT, NT)] += prod

                    @pl.when(n + 2 < N_TILES)
                    def _():
                        wdma_start(s, n + 2, slot)

                    return 0

                lax.fori_loop(0, N_TILES, jh_body, 0)

        s3 = src_of(3)
        for q in (0, 1, 2, 3):
            wdma_start(s3, 0, 0)
            wdma_start(s3, 1, 1)
            wait_recv(3, q, q * MQ, MQ)
            xq = pltpu.make_async_copy(
                xg_ref.at[s3, pl.ds(q * MQ, MQ), :], cbf, x_sem
            )
            xq.start()
            xq.wait()
            row = slice(q * MQ, (q + 1) * MQ)

            def jq_body(n, _, row=row):
                slot = lax.rem(n, 2)
                wdma_wait(slot)
                prod = jnp.dot(
                    cbf[:, :], w_buf[slot].astype(jnp.bfloat16),
                    preferred_element_type=jnp.float32,
                )
                acc_ref[row, pl.ds(n * NT, NT)] += prod

                @pl.when(n + 2 < N_TILES)
                def _():
                    wdma_start(s3, n + 2, slot)

                return 0

            lax.fori_loop(0, N_TILES, jq_body, 0)

            def silu_body(n, _, row=row):
                v = acc_ref[row, pl.ds(n * NT, NT)]
                acc_ref[row, pl.ds(n * NT, NT)] = v * (
                    1.0 / (1.0 + jnp.exp(-v))
                )
                return 0

            lax.fori_loop(0, N_TILES, silu_body, 0)
            pltpu.make_async_copy(
                acc_ref.at[row, :], out_ref.at[row, :], out_sems.at[q]
            ).start()

        for q in (0, 1, 2, 3):
            pltpu.make_async_copy(
                acc_ref.at[pl.ds(q * MQ, MQ), :],
                out_ref.at[pl.ds(q * MQ, MQ), :],
                out_sems.at[q],
            ).wait()

        for rdma in sends:
            rdma.wait_send()

    y, _, _ = pl.pallas_call(
        body,
        out_shape=[
            jax.ShapeDtypeStruct((M_BLK, n_out), jnp.float32),
            jax.ShapeDtypeStruct((N_DEV, M_BLK, k_loc), jnp.bfloat16),
            jax.ShapeDtypeStruct((N_DEV - 1, M_BLK, k_loc), jnp.bfloat16),
        ],
        in_specs=[
            pl.BlockSpec(memory_space=pl.ANY),
            pl.BlockSpec(memory_space=pl.ANY),
        ],
        out_specs=[
            pl.BlockSpec(memory_space=pl.ANY),
            pl.BlockSpec(memory_space=pl.ANY),
            pl.BlockSpec(memory_space=pl.ANY),
        ],
        scratch_shapes=[
            pltpu.VMEM((MQ, k_loc), jnp.float32),
            pltpu.VMEM((MQ, k_loc), jnp.bfloat16),
            pltpu.VMEM((MH, k_loc), jnp.float32),
            pltpu.VMEM((MH, k_loc), jnp.bfloat16),
            pltpu.VMEM((2, M_BLK, NT), jnp.float32),
            pltpu.VMEM((M_BLK, n_out), jnp.float32),
            pltpu.SemaphoreType.DMA((N_DEV - 1, 4)),
            pltpu.SemaphoreType.DMA((N_DEV - 1, 4)),
            pltpu.SemaphoreType.DMA,
            pltpu.SemaphoreType.DMA,
            pltpu.SemaphoreType.DMA,
            pltpu.SemaphoreType.DMA((2,)),
            pltpu.SemaphoreType.DMA((4,)),
        ],
        compiler_params=pltpu.CompilerParams(
            collective_id=0,
            vmem_limit_bytes=63 * 1024 * 1024,
        ),
    )(x, w_mat)
    return y


# device time: 325640 ns/iter; 1.0278x vs baseline; 1.0278x over previous
import jax
import jax.numpy as jnp
from jax import lax
from jax.experimental import pallas as pl
from jax.experimental.pallas import tpu as pltpu

jax.config.update("jax_compilation_cache_dir", "/tmp/jax_cache")
jax.config.update("jax_persistent_cache_min_compile_time_secs", 0.0)

N_DEV = 4
M_BLK = 2048
MH = 1024
MQ = 512
NT = 512
N_TILES = 8

_SRC_OFF = (0, 3, 1, 2)
_SEM_FOR = (None, 0, 2, 1)


def kernel(x, w_mat):
    k_tot, k_loc = x.shape
    n_out = w_mat.shape[1]
    assert k_loc == M_BLK and n_out == N_TILES * NT

    def body(x_ref, w_ref, out_ref, xg_ref, stage_ref,
             cf32, cbf, xh_f32, x_half, w_buf, acc_ref,
             send_sems, recv_sems, cin_sem, cout_sem, x_sem, w_sems,
             out_sems):
        my = lax.axis_index("i")

        def src_of(j):
            return lax.rem(my + _SRC_OFF[j], N_DEV)

        barrier = pltpu.get_barrier_semaphore()
        for d in (1, 2, 3):
            t = lax.rem(my + d, N_DEV)
            pl.semaphore_signal(
                barrier, inc=1, device_id=(t,),
                device_id_type=pl.DeviceIdType.MESH,
            )
        pl.semaphore_wait(barrier, N_DEV - 1)

        sends = []
        for d in (1, 2, 3):
            t = lax.rem(my + d, N_DEV)
            for h in (0, 1):
                for qh in (0, 1):
                    q = 2 * h + qh
                    cin = pltpu.make_async_copy(
                        x_ref.at[pl.ds(t * M_BLK + q * MQ, MQ), :],
                        cf32, cin_sem,
                    )
                    cin.start()
                    cin.wait()
                    cbf[:, :] = cf32[:, :].astype(jnp.bfloat16)
                    cout = pltpu.make_async_copy(
                        cbf, stage_ref.at[d - 1, pl.ds(q * MQ, MQ), :],
                        cout_sem,
                    )
                    cout.start()
                    cout.wait()
                rdma = pltpu.make_async_remote_copy(
                    src_ref=stage_ref.at[d - 1, pl.ds(h * MH, MH), :],
                    dst_ref=xg_ref.at[my, pl.ds(h * MH, MH), :],
                    send_sem=send_sems.at[d - 1, h],
                    recv_sem=recv_sems.at[d - 1, h],
                    device_id=(t,),
                    device_id_type=pl.DeviceIdType.MESH,
                )
                rdma.start()
                sends.append(rdma)

        def wait_recv(j, h):
            s = src_of(j)
            sem = _SEM_FOR[j]
            pltpu.make_async_remote_copy(
                src_ref=stage_ref.at[0, pl.ds(h * MH, MH), :],
                dst_ref=xg_ref.at[s, pl.ds(h * MH, MH), :],
                send_sem=send_sems.at[sem, h],
                recv_sem=recv_sems.at[sem, h],
                device_id=(s,),
                device_id_type=pl.DeviceIdType.MESH,
            ).wait_recv()

        def wdma_start(s, n, slot):
            pltpu.make_async_copy(
                w_ref.at[pl.ds(s * M_BLK, M_BLK), pl.ds(n * NT, NT)],
                w_buf.at[slot], w_sems.at[slot],
            ).start()

        def wdma_wait(slot):
            pltpu.make_async_copy(
                w_ref.at[pl.ds(0, M_BLK), pl.ds(0, NT)],
                w_buf.at[slot], w_sems.at[slot],
            ).wait()

        s0 = src_of(0)
        for h in (0, 1):
            xh = pltpu.make_async_copy(
                x_ref.at[pl.ds(my * M_BLK + h * MH, MH), :], xh_f32, x_sem
            )
            xh.start()
            wdma_start(s0, 0, 0)
            wdma_start(s0, 1, 1)
            xh.wait()
            row = slice(h * MH, (h + 1) * MH)

            def j0_body(n, _, row=row):
                slot = lax.rem(n, 2)
                wdma_wait(slot)
                prod = jnp.dot(
                    xh_f32[:, :], w_buf[slot],
                    preferred_element_type=jnp.float32,
                )
                acc_ref[row, pl.ds(n * NT, NT)] = prod

                @pl.when(n + 2 < N_TILES)
                def _():
                    wdma_start(s0, n + 2, slot)

                return 0

            lax.fori_loop(0, N_TILES, j0_body, 0)

        for j in (1, 2, 3):
            s = src_of(j)
            for h in (0, 1):
                wdma_start(s, 0, 0)
                wdma_start(s, 1, 1)
                wait_recv(j, h)
                xh = pltpu.make_async_copy(
                    xg_ref.at[s, pl.ds(h * MH, MH), :], x_half, x_sem
                )
                xh.start()
                xh.wait()
                row = slice(h * MH, (h + 1) * MH)

                def jh_body(n, _, j=j, s=s, row=row):
                    slot = lax.rem(n, 2)
                    wdma_wait(slot)
                    prod = jnp.dot(
                        x_half[:, :], w_buf[slot].astype(jnp.bfloat16),
                        preferred_element_type=jnp.float32,
                    )
                    acc_ref[row, pl.ds(n * NT, NT)] += prod

                    @pl.when(n + 2 < N_TILES)
                    def _():
                        wdma_start(s, n + 2, slot)

                    return 0

                lax.fori_loop(0, N_TILES, jh_body, 0)

                if j == 3:
                    def silu_body(n, _, row=row):
                        v = acc_ref[row, pl.ds(n * NT, NT)]
                        acc_ref[row, pl.ds(n * NT, NT)] = v * (
                            1.0 / (1.0 + jnp.exp(-v))
                        )
                        return 0

                    lax.fori_loop(0, N_TILES, silu_body, 0)
                    pltpu.make_async_copy(
                        acc_ref.at[row, :], out_ref.at[row, :], out_sems.at[h]
                    ).start()

        for h in (0, 1):
            pltpu.make_async_copy(
                acc_ref.at[pl.ds(h * MH, MH), :],
                out_ref.at[pl.ds(h * MH, MH), :],
                out_sems.at[h],
            ).wait()

        for rdma in sends:
            rdma.wait_send()

    y, _, _ = pl.pallas_call(
        body,
        out_shape=[
            jax.ShapeDtypeStruct((M_BLK, n_out), jnp.float32),
            jax.ShapeDtypeStruct((N_DEV, M_BLK, k_loc), jnp.bfloat16),
            jax.ShapeDtypeStruct((N_DEV - 1, M_BLK, k_loc), jnp.bfloat16),
        ],
        in_specs=[
            pl.BlockSpec(memory_space=pl.ANY),
            pl.BlockSpec(memory_space=pl.ANY),
        ],
        out_specs=[
            pl.BlockSpec(memory_space=pl.ANY),
            pl.BlockSpec(memory_space=pl.ANY),
            pl.BlockSpec(memory_space=pl.ANY),
        ],
        scratch_shapes=[
            pltpu.VMEM((MQ, k_loc), jnp.float32),
            pltpu.VMEM((MQ, k_loc), jnp.bfloat16),
            pltpu.VMEM((MH, k_loc), jnp.float32),
            pltpu.VMEM((MH, k_loc), jnp.bfloat16),
            pltpu.VMEM((2, M_BLK, NT), jnp.float32),
            pltpu.VMEM((M_BLK, n_out), jnp.float32),
            pltpu.SemaphoreType.DMA((N_DEV - 1, 2)),
            pltpu.SemaphoreType.DMA((N_DEV - 1, 2)),
            pltpu.SemaphoreType.DMA,
            pltpu.SemaphoreType.DMA,
            pltpu.SemaphoreType.DMA,
            pltpu.SemaphoreType.DMA((2,)),
            pltpu.SemaphoreType.DMA((2,)),
        ],
        compiler_params=pltpu.CompilerParams(
            collective_id=0,
            vmem_limit_bytes=63 * 1024 * 1024,
        ),
    )(x, w_mat)
    return y
